# baseline (speedup 1.0000x reference)
import functools

import jax
import jax.numpy as jnp
from jax import lax
from jax.experimental import pallas as pl
from jax.experimental.pallas import tpu as pltpu
from jax.experimental.pallas import tpu_sc as plsc

_D = 32
_G = 128
_B_TOT = 16384 * 50
_N_GROUPS = _B_TOT // _G
_NC = 2
_NS = 16
_NW = _NC * _NS
_G_PER_W = _N_GROUPS // _NW

_mesh = plsc.VectorSubcoreMesh(core_axis_name="c", subcore_axis_name="s")


@functools.partial(
    pl.kernel,
    out_type=jax.ShapeDtypeStruct((_B_TOT, _D), jnp.float32),
    mesh=_mesh,
    compiler_params=pltpu.CompilerParams(use_tc_tiling_on_sc=False),
    scratch_types=[
        pltpu.VMEM((_G,), jnp.int32),
        pltpu.VMEM((_G, _D), jnp.float32),
        pltpu.SemaphoreType.DMA,
    ],
)
def _gather_kernel(idx_hbm, table_hbm, out_hbm, idx_v, rows_v, sem):
    wid = lax.axis_index("s") * _NC + lax.axis_index("c")
    g_base = wid * _G_PER_W

    def body(g, carry):
        pltpu.sync_copy(idx_hbm.at[g_base + g], idx_v)
        pltpu.async_copy(table_hbm.at[idx_v], rows_v, sem).wait()
        pltpu.sync_copy(rows_v, out_hbm.at[pl.ds((g_base + g) * _G, _G)])
        return carry

    lax.fori_loop(0, _G_PER_W, body, 0)


def kernel(x, emb_weight):
    idx = x.reshape(_N_GROUPS, _G).astype(jnp.int32)
    out = _gather_kernel(idx, emb_weight)
    return out.reshape(x.shape + (emb_weight.shape[1],))


# speedup vs baseline: 1.0975x; 1.0975x over previous
import functools

import jax
import jax.numpy as jnp
from jax import lax
from jax.experimental import pallas as pl
from jax.experimental.pallas import tpu as pltpu
from jax.experimental.pallas import tpu_sc as plsc

_D = 32
_G = 128
_B_TOT = 16384 * 50
_N_GROUPS = _B_TOT // _G
_NC = 2
_NS = 16
_NW = _NC * _NS
_G_PER_W = _N_GROUPS // _NW
_mesh = plsc.VectorSubcoreMesh(core_axis_name="c", subcore_axis_name="s")


@functools.partial(
    pl.kernel,
    out_type=jax.ShapeDtypeStruct((_B_TOT, _D), jnp.float32),
    mesh=_mesh,
    compiler_params=pltpu.CompilerParams(use_tc_tiling_on_sc=False),
    scratch_types=[
        pltpu.VMEM((_G_PER_W, _G), jnp.int32),
        pltpu.VMEM((2, _G, _D), jnp.float32),
        pltpu.SemaphoreType.DMA,
        pltpu.SemaphoreType.DMA((2,)),
        pltpu.SemaphoreType.DMA((2,)),
    ],
)
def _gather_kernel(idx_hbm, table_hbm, out_hbm, idx_v, rows_v,
                   sem_i, sem_g, sem_o):
    wid = lax.axis_index("s") * _NC + lax.axis_index("c")
    g_base = wid * _G_PER_W

    pltpu.async_copy(
        idx_hbm.at[pl.ds(g_base, _G_PER_W)], idx_v, sem_i).wait()

    def gather_cp(g, j):
        return pltpu.make_async_copy(
            table_hbm.at[idx_v.at[g]], rows_v.at[j], sem_g.at[j])

    def out_cp(g, j):
        return pltpu.make_async_copy(
            rows_v.at[j], out_hbm.at[pl.ds((g_base + g) * _G, _G)],
            sem_o.at[j])

    def body(i, carry):
        g0 = 2 * i

        @pl.when(i > 0)
        def _():
            out_cp(g0 - 2, 0).wait()
            out_cp(g0 - 1, 1).wait()
        gather_cp(g0, 0).start()
        gather_cp(g0 + 1, 1).start()
        gather_cp(g0, 0).wait()
        out_cp(g0, 0).start()
        gather_cp(g0 + 1, 1).wait()
        out_cp(g0 + 1, 1).start()
        return carry

    lax.fori_loop(0, _G_PER_W // 2, body, 0)
    out_cp(_G_PER_W - 2, 0).wait()
    out_cp(_G_PER_W - 1, 1).wait()


def kernel(x, emb_weight):
    idx = x.reshape(_N_GROUPS, _G).astype(jnp.int32)
    out = _gather_kernel(idx, emb_weight)
    return out.reshape(x.shape + (emb_weight.shape[1],))


# speedup vs baseline: 1.1163x; 1.0171x over previous
import functools

import jax
import jax.numpy as jnp
from jax import lax
from jax.experimental import pallas as pl
from jax.experimental.pallas import tpu as pltpu
from jax.experimental.pallas import tpu_sc as plsc

_D = 32
_G = 128
_B_TOT = 16384 * 50
_N_GROUPS = _B_TOT // _G
_NC = 2
_NS = 16
_NW = _NC * _NS
_G_PER_W = _N_GROUPS // _NW
_NB = 4
_KG = 2

_mesh = plsc.VectorSubcoreMesh(core_axis_name="c", subcore_axis_name="s")


@functools.partial(
    pl.kernel,
    out_type=jax.ShapeDtypeStruct((_B_TOT, _D), jnp.float32),
    mesh=_mesh,
    compiler_params=pltpu.CompilerParams(use_tc_tiling_on_sc=False),
    scratch_types=[
        pltpu.VMEM((_G_PER_W, _G), jnp.int32),
        pltpu.VMEM((_NB, _G, _D), jnp.float32),
        pltpu.SemaphoreType.DMA,
        pltpu.SemaphoreType.DMA((_NB,)),
        pltpu.SemaphoreType.DMA((_NB,)),
    ],
)
def _gather_kernel(idx_hbm, table_hbm, out_hbm, idx_v, rows_v,
                   sem_i, sem_g, sem_o):
    wid = lax.axis_index("s") * _NC + lax.axis_index("c")
    g_base = wid * _G_PER_W

    pltpu.async_copy(
        idx_hbm.at[pl.ds(g_base, _G_PER_W)], idx_v, sem_i).wait()

    def gather_cp(g, j):
        return pltpu.make_async_copy(
            table_hbm.at[idx_v.at[g]], rows_v.at[j], sem_g.at[j])

    def out_cp(g, j):
        return pltpu.make_async_copy(
            rows_v.at[j], out_hbm.at[pl.ds((g_base + g) * _G, _G)],
            sem_o.at[j])

    def body(i, carry):
        g0 = _NB * i

        for j in range(_NB):
            @pl.when(i > 0)
            def _(j=j):
                out_cp(g0 - _NB + j, j).wait()
            gather_cp(g0 + j, j).start()
            if j >= _KG:
                jk = j - _KG
                gather_cp(g0 + jk, jk).wait()
                out_cp(g0 + jk, jk).start()
        for j in range(_NB - _KG, _NB):
            gather_cp(g0 + j, j).wait()
            out_cp(g0 + j, j).start()
        return carry

    lax.fori_loop(0, _G_PER_W // _NB, body, 0)
    for j in range(_NB):
        out_cp(_G_PER_W - _NB + j, j).wait()


def kernel(x, emb_weight):
    idx = x.reshape(_N_GROUPS, _G).astype(jnp.int32)
    out = _gather_kernel(idx, emb_weight)
    return out.reshape(x.shape + (emb_weight.shape[1],))


# speedup vs baseline: 1.1394x; 1.0207x over previous
import functools

import jax
import jax.numpy as jnp
from jax import lax
from jax.experimental import pallas as pl
from jax.experimental.pallas import tpu as pltpu
from jax.experimental.pallas import tpu_sc as plsc

_D = 32
_B_TOT = 16384 * 50
_SZ = 512
_N_STREAMS = _B_TOT // _SZ
_NC = 2
_NS = 16
_NW = _NC * _NS
_S_PER_W = _N_STREAMS // _NW
_NB = 5
_KG = 2
_NBLK = _S_PER_W // _NB

_mesh = plsc.VectorSubcoreMesh(core_axis_name="c", subcore_axis_name="s")


@functools.partial(
    pl.kernel,
    out_type=jax.ShapeDtypeStruct((_B_TOT, _D), jnp.float32),
    mesh=_mesh,
    compiler_params=pltpu.CompilerParams(use_tc_tiling_on_sc=False),
    scratch_types=(
        [pltpu.VMEM((_SZ,), jnp.int32) for _ in range(_NB)]
        + [
            pltpu.VMEM((_NB, _SZ, _D), jnp.float32),
            pltpu.SemaphoreType.DMA((_NB,)),
            pltpu.SemaphoreType.DMA((_NB,)),
            pltpu.SemaphoreType.DMA((_NB,)),
        ]
    ),
)
def _gather_kernel(idx_hbm, table_hbm, out_hbm, *scr):
    idx_b = scr[:_NB]
    rows_v, sem_i, sem_g, sem_o = scr[_NB:]
    wid = lax.axis_index("s") * _NC + lax.axis_index("c")
    s_base = wid * _S_PER_W

    def idx_cp(s, j):
        return pltpu.make_async_copy(
            idx_hbm.at[pl.ds((s_base + s) * _SZ, _SZ)], idx_b[j],
            sem_i.at[j])

    def gather_cp(j):
        return pltpu.make_async_copy(
            table_hbm.at[idx_b[j]], rows_v.at[j], sem_g.at[j])

    def out_cp(s, j):
        return pltpu.make_async_copy(
            rows_v.at[j],
            out_hbm.at[pl.ds((s_base + s) * _SZ, _SZ)],
            sem_o.at[j])

    for j in range(_NB):
        idx_cp(j, j).start()

    def body(i, carry):
        s0 = _NB * i
        for j in range(_NB):
            s = s0 + j
            idx_cp(s, j).wait()

            @pl.when(i > 0)
            def _(j=j, s=s):
                out_cp(s - _NB, j).wait()
            gather_cp(j).start()
            if j >= _KG:
                jk = j - _KG
                sk = s0 + jk
                gather_cp(jk).wait()
                out_cp(sk, jk).start()

                @pl.when(i < _NBLK - 1)
                def _(jk=jk, sk=sk):
                    idx_cp(sk + _NB, jk).start()
        for j in range(_NB - _KG, _NB):
            s = s0 + j
            gather_cp(j).wait()
            out_cp(s, j).start()

            @pl.when(i < _NBLK - 1)
            def _(j=j, s=s):
                idx_cp(s + _NB, j).start()
        return carry

    lax.fori_loop(0, _NBLK, body, 0)
    for j in range(_NB):
        out_cp(_S_PER_W - _NB + j, j).wait()


def kernel(x, emb_weight):
    idx = x.reshape(-1).astype(jnp.int32)
    out = _gather_kernel(idx, emb_weight)
    return out.reshape(x.shape + (emb_weight.shape[1],))


# speedup vs baseline: 1.6267x; 1.4277x over previous
import functools

import jax
import jax.numpy as jnp
from jax import lax
from jax.experimental import pallas as pl
from jax.experimental.pallas import tpu as pltpu
from jax.experimental.pallas import tpu_sc as plsc

_D = 32
_B = 16384
_H = 50
_NC = 2
_NS = 16
_NW = _NC * _NS
_B_PER_W = _B // _NW
_HC = 5
_NHC = _H // _HC
_NBB = _B_PER_W // 128
_SZ = _HC * 128
_S_PER_W = _NHC * _NBB
_NB = 2

_mesh = plsc.VectorSubcoreMesh(core_axis_name="c", subcore_axis_name="s")


@functools.partial(
    pl.kernel,
    out_type=jax.ShapeDtypeStruct((_H, _D // 8, _B // 128, 8, 128),
                                  jnp.float32),
    mesh=_mesh,
    compiler_params=pltpu.CompilerParams(use_tc_tiling_on_sc=False,
                                         needs_layout_passes=False),
    scratch_types=(
        [pltpu.VMEM((_SZ,), jnp.int32) for _ in range(_NB)]
        + [
            pltpu.VMEM((_B_PER_W * _H,), jnp.int32),
            pltpu.VMEM((_NB, _SZ, _D), jnp.float32),
            pltpu.VMEM((_NB, _HC, _D // 8, 8, 128), jnp.float32),
            pltpu.SemaphoreType.DMA,
            pltpu.SemaphoreType.DMA((_NB,)),
            pltpu.SemaphoreType.DMA((_NB,)),
        ]
    ),
)
def _gather_kernel(idx_hbm, table_hbm, out_hbm, *scr):
    off_b = scr[:_NB]
    idx_v, rows_v, tv, sem_i, sem_g, sem_o = scr[_NB:]
    wid = lax.axis_index("s") * _NC + lax.axis_index("c")

    pltpu.async_copy(
        idx_hbm.at[pl.ds(wid * _B_PER_W * _H, _B_PER_W * _H)], idx_v,
        sem_i).wait()

    lanes = lax.broadcasted_iota(jnp.int32, (16,), 0)

    def gather_cp(j):
        return pltpu.make_async_copy(
            table_hbm.at[off_b[j]], rows_v.at[j], sem_g.at[j])

    def out_cp(s, j):
        hc = s // _NBB
        bbg = wid * _NBB + lax.rem(s, _NBB)
        return pltpu.make_async_copy(
            tv.at[j],
            out_hbm.at[pl.ds(hc * _HC, _HC), pl.ds(0, _D // 8), bbg],
            sem_o.at[j])

    def build_offsets(s, j):
        hc = s // _NBB
        bb = lax.rem(s, _NBB)
        base = bb * 128 * _H + hc * _HC
        for ho in range(_HC):
            for bl0 in range(0, 128, 16):
                src = (bl0 + lanes) * _H + (base + ho)
                off_b[j][pl.ds(ho * 128 + bl0, 16)] = plsc.load_gather(
                    idx_v, [src])

    def transpose(j):
        def t_body(t, carry):
            ho = t // (_D // 8)
            fb = lax.rem(t, _D // 8)
            for fs in range(8):
                for bl0 in range(0, 128, 16):
                    rows = ho * 128 + bl0 + lanes
                    cols = jnp.full((16,), fb * 8 + fs, jnp.int32)
                    tv[j, ho, fb, fs, pl.ds(bl0, 16)] = plsc.load_gather(
                        rows_v.at[j], [rows, cols])
            return carry

        lax.fori_loop(0, _HC * (_D // 8), t_body, 0)

    def body(i, carry):
        s0 = _NB * i
        for j in range(_NB):
            @pl.when(i > 0)
            def _(j=j, s=s0 + j):
                out_cp(s - _NB, j).wait()
            build_offsets(s0 + j, j)
            gather_cp(j).start()
        for j in range(_NB):
            gather_cp(j).wait()
            transpose(j)
            out_cp(s0 + j, j).start()
        return carry

    lax.fori_loop(0, _S_PER_W // _NB, body, 0)
    for j in range(_NB):
        out_cp(_S_PER_W - _NB + j, j).wait()


def kernel(x, emb_weight):
    idx = x.reshape(-1).astype(jnp.int32)
    out = _gather_kernel(idx, emb_weight)
    out = out.transpose(2, 4, 0, 1, 3)
    return out.reshape(x.shape + (emb_weight.shape[1],))


# speedup vs baseline: 2.1297x; 1.3092x over previous
import functools

import jax
import jax.numpy as jnp
from jax import lax
from jax.experimental import pallas as pl
from jax.experimental.pallas import tpu as pltpu
from jax.experimental.pallas import tpu_sc as plsc

_D = 32
_B = 16384
_H = 50
_NC = 2
_NS = 16
_NW = _NC * _NS
_B_PER_W = _B // _NW
_HC = 5
_NHC = _H // _HC
_NBB = _B_PER_W // 128
_SZ = _HC * 128
_S_PER_W = _NHC * _NBB
_NB = 2

_mesh = plsc.VectorSubcoreMesh(core_axis_name="c", subcore_axis_name="s")


@functools.partial(
    pl.kernel,
    out_type=jax.ShapeDtypeStruct((_H, _D // 8, _B // 128, 8, 128),
                                  jnp.float32),
    mesh=_mesh,
    compiler_params=pltpu.CompilerParams(use_tc_tiling_on_sc=False,
                                         needs_layout_passes=False),
    scratch_types=(
        [pltpu.VMEM((_SZ,), jnp.int32) for _ in range(_NB)]
        + [
            pltpu.VMEM((_B_PER_W * _H,), jnp.int32),
            pltpu.VMEM((_NB, _SZ, _D), jnp.float32),
            pltpu.VMEM((_NB, _HC, _D // 8, 8, 128), jnp.float32),
            pltpu.SemaphoreType.DMA,
            pltpu.SemaphoreType.DMA((_NB,)),
            pltpu.SemaphoreType.DMA((_NB,)),
        ]
    ),
)
def _gather_kernel(idx_hbm, table_hbm, out_hbm, *scr):
    off_b = scr[:_NB]
    idx_v, rows_v, tv, sem_i, sem_g, sem_o = scr[_NB:]
    wid = lax.axis_index("s") * _NC + lax.axis_index("c")

    pltpu.async_copy(
        idx_hbm.at[pl.ds(wid * _B_PER_W * _H, _B_PER_W * _H)], idx_v,
        sem_i).wait()

    lanes = lax.broadcasted_iota(jnp.int32, (16,), 0)

    def gather_cp(j):
        return pltpu.make_async_copy(
            table_hbm.at[off_b[j]], rows_v.at[j], sem_g.at[j])

    def out_cp(s, j):
        hc = s // _NBB
        bbg = wid * _NBB + lax.rem(s, _NBB)
        return pltpu.make_async_copy(
            tv.at[j],
            out_hbm.at[pl.ds(hc * _HC, _HC), pl.ds(0, _D // 8), bbg],
            sem_o.at[j])

    def build_offsets(s, j):
        hc = s // _NBB
        bb = lax.rem(s, _NBB)
        base = bb * 128 * _H + hc * _HC
        for ho in range(_HC):
            for bl0 in range(0, 128, 16):
                src = (bl0 + lanes) * _H + (base + ho)
                off_b[j][pl.ds(ho * 128 + bl0, 16)] = plsc.load_gather(
                    idx_v, [src])

    def transpose(j):
        @plsc.parallel_loop(0, _HC * (_D // 8), unroll=4)
        def t_body(t):
            ho = t // (_D // 8)
            fb = lax.rem(t, _D // 8)
            for fs in range(8):
                for bl0 in range(0, 128, 16):
                    rows = ho * 128 + bl0 + lanes
                    cols = jnp.full((16,), fb * 8 + fs, jnp.int32)
                    tv[j, ho, fb, fs, pl.ds(bl0, 16)] = plsc.load_gather(
                        rows_v.at[j], [rows, cols])

    def body(i, carry):
        s0 = _NB * i
        for j in range(_NB):
            @pl.when(i > 0)
            def _(j=j, s=s0 + j):
                out_cp(s - _NB, j).wait()
            build_offsets(s0 + j, j)
            gather_cp(j).start()
        for j in range(_NB):
            gather_cp(j).wait()
            transpose(j)
            out_cp(s0 + j, j).start()
        return carry

    lax.fori_loop(0, _S_PER_W // _NB, body, 0)
    for j in range(_NB):
        out_cp(_S_PER_W - _NB + j, j).wait()


def kernel(x, emb_weight):
    idx = x.reshape(-1).astype(jnp.int32)
    out = _gather_kernel(idx, emb_weight)
    out = out.transpose(2, 4, 0, 1, 3)
    return out.reshape(x.shape + (emb_weight.shape[1],))
